# Initial kernel scaffold; baseline (speedup 1.0000x reference)
import jax
import jax.numpy as jnp
from jax.experimental import pallas as pl


def kernel(score):
    raise NotImplementedError("write your pallas kernel here")


# speedup vs baseline: 1.9338x; 1.9338x over previous
import functools

import jax
import jax.numpy as jnp
from jax import lax
from jax.experimental import pallas as pl
from jax.experimental.pallas import tpu as pltpu
from jax.experimental.pallas import tpu_sc as plsc

BATCH = 64
BEAM = 8
VOCAB = 100000
ROW = BEAM * VOCAB
K = 8
L = 16
NC, NS = 2, 16
NW = NC * NS
ROWS_PER_W = BATCH // NW
CHUNK = 32000
NCHUNK = ROW // CHUNK
GVREGS = 16
GROUP = GVREGS * L
NGROUP = CHUNK // GROUP

NEG_INF = float("-inf")
I32_MAX = 2**31 - 1


def _bubble_insert(v, iv, vals, idxs):
    vals = list(vals)
    idxs = list(idxs)
    nv, ni = v, iv
    for lvl in range(K):
        tv, ti = vals[lvl], idxs[lvl]
        take = (nv > tv) | ((nv == tv) & (ni < ti))
        vals[lvl] = jnp.where(take, nv, tv)
        idxs[lvl] = jnp.where(take, ni, ti)
        nv = jnp.where(take, tv, nv)
        ni = jnp.where(take, ti, ni)
    return tuple(vals), tuple(idxs)


def _tec_body(score_hbm, vals_hbm, rows_hbm, cols_hbm, buf, ov_ref, or_ref, oc_ref):
    wid = lax.axis_index("s") * NC + lax.axis_index("c")
    lane = lax.iota(jnp.int32, L)

    out_v = jnp.full((L,), 0.0, jnp.float32)
    out_i = jnp.full((L,), 0, jnp.int32)

    for rr in range(ROWS_PER_W):
        row = wid * ROWS_PER_W + rr
        row_base = row * ROW

        vals = tuple(jnp.full((L,), NEG_INF, jnp.float32) for _ in range(K))
        idxs = tuple(jnp.full((L,), 0, jnp.int32) for _ in range(K))
        s_t = NEG_INF

        def chunk_body(c, carry):
            vals, idxs, s_t = carry
            pltpu.sync_copy(score_hbm.at[pl.ds(row_base + c * CHUNK, CHUNK)], buf)

            def group_body(g, carry):
                vals, idxs, s_t = carry
                off = g * GROUP
                vs = [buf[pl.ds(off + i * L, L)] for i in range(GVREGS)]
                gm = vs[0]
                for i in range(1, GVREGS):
                    gm = jnp.maximum(gm, vs[i])
                gmax = jnp.max(gm)

                def slow(vals, idxs):
                    ebase = c * CHUNK + off
                    for i in range(GVREGS):
                        v = vs[i]
                        hit = jnp.max(jnp.where(v > vals[K - 1], jnp.int32(1),
                                                jnp.int32(0)))
                        iv = lane + (ebase + i * L)
                        vals, idxs = lax.cond(
                            hit > 0,
                            lambda vl, ix, v=v, iv=iv: _bubble_insert(v, iv, vl, ix),
                            lambda vl, ix: (vl, ix),
                            vals, idxs)
                    return vals, idxs, jnp.min(vals[K - 1])

                return lax.cond(gmax > s_t,
                                lambda: slow(vals, idxs),
                                lambda: (vals, idxs, s_t))

            return lax.fori_loop(0, NGROUP, group_body, (vals, idxs, s_t))

        vals, idxs, s_t = lax.fori_loop(0, NCHUNK, chunk_body, (vals, idxs, s_t))

        vals = list(vals)
        for p in range(K):
            mv = vals[0]
            for j in range(1, K):
                mv = jnp.maximum(mv, vals[j])
            m = jnp.max(mv)
            iw = [jnp.where(vals[j] == m, idxs[j], I32_MAX) for j in range(K)]
            mi = iw[0]
            for j in range(1, K):
                mi = jnp.minimum(mi, iw[j])
            mi = jnp.min(mi)
            for j in range(K):
                vals[j] = jnp.where((vals[j] == m) & (idxs[j] == mi),
                                    NEG_INF, vals[j])
            sel = lane == (rr * K + p)
            out_v = jnp.where(sel, m, out_v)
            out_i = jnp.where(sel, mi, out_i)

    out_r = out_i // VOCAB
    out_c = out_i - out_r * VOCAB
    ov_ref[...] = out_v
    or_ref[...] = out_r
    oc_ref[...] = out_c
    pltpu.sync_copy(ov_ref, vals_hbm.at[pl.ds(wid * L, L)])
    pltpu.sync_copy(or_ref, rows_hbm.at[pl.ds(wid * L, L)])
    pltpu.sync_copy(oc_ref, cols_hbm.at[pl.ds(wid * L, L)])


@jax.jit
def kernel(score):
    flat = score.reshape(BATCH * ROW)
    mesh = plsc.VectorSubcoreMesh(core_axis_name="c", subcore_axis_name="s",
                                  num_cores=NC, num_subcores=NS)
    vals, rows, cols = pl.kernel(
        _tec_body,
        out_type=(
            jax.ShapeDtypeStruct((BATCH * K,), jnp.float32),
            jax.ShapeDtypeStruct((BATCH * K,), jnp.int32),
            jax.ShapeDtypeStruct((BATCH * K,), jnp.int32),
        ),
        mesh=mesh,
        compiler_params=pltpu.CompilerParams(needs_layout_passes=False),
        scratch_types=[
            pltpu.VMEM((CHUNK,), jnp.float32),
            pltpu.VMEM((L,), jnp.float32),
            pltpu.VMEM((L,), jnp.int32),
            pltpu.VMEM((L,), jnp.int32),
        ],
    )(flat)
    return (vals.reshape(BATCH, K), rows.reshape(BATCH, K),
            cols.reshape(BATCH, K))
